# initial kernel scaffold (unmeasured)
import jax
import jax.numpy as jnp
from jax import lax
from jax.experimental import pallas as pl
from jax.experimental.pallas import tpu as pltpu

N_DEV = 8


def kernel(x, k, Wp):
    B, S, C = x.shape
    T = k.shape[0]
    P = Wp.shape[1]

    def body(x_ref, k_ref, w_ref, out_ref, acc_ref, comm_ref, send_sems, recv_sems):
        my = lax.axis_index("i")
        right = lax.rem(my + 1, N_DEV)

        xv = x_ref[:, :, :]
        kv = k_ref[:, :]
        conv = xv * kv[T - 1][None, None, :]
        for t in range(T - 1):
            d = T - 1 - t
            shifted = jnp.concatenate(
                [jnp.zeros((B, d, C), dtype=xv.dtype), xv[:, : S - d, :]],
                axis=1,
            )
            conv = conv + shifted * kv[t][None, None, :]
        a = conv * jax.nn.sigmoid(conv)
        a2 = a.reshape(B * S, C).astype(jnp.bfloat16)
        w = w_ref[:, :].astype(jnp.bfloat16)
        partial = jnp.dot(a2, w, preferred_element_type=jnp.float32)

        acc_ref[:, :] = partial
        comm_ref[0, :, :] = partial.astype(jnp.bfloat16)

        for h in range(N_DEV - 1):
            rdma = pltpu.make_async_remote_copy(
                src_ref=comm_ref.at[h],
                dst_ref=comm_ref.at[h + 1],
                send_sem=send_sems.at[h],
                recv_sem=recv_sems.at[h],
                device_id=(right,),
                device_id_type=pl.DeviceIdType.MESH,
            )
            rdma.start()
            rdma.wait()
            acc_ref[:, :] = acc_ref[:, :] + comm_ref[h + 1, :, :].astype(jnp.float32)

        out_ref[:, :, :] = acc_ref[:, :].reshape(B, S, P)

    return pl.pallas_call(
        body,
        out_shape=jax.ShapeDtypeStruct((B, S, P), jnp.float32),
        in_specs=[
            pl.BlockSpec(memory_space=pltpu.VMEM),
            pl.BlockSpec(memory_space=pltpu.VMEM),
            pl.BlockSpec(memory_space=pltpu.VMEM),
        ],
        out_specs=pl.BlockSpec(memory_space=pltpu.VMEM),
        scratch_shapes=[
            pltpu.VMEM((B * S, P), jnp.float32),
            pltpu.VMEM((N_DEV, B * S, P), jnp.bfloat16),
            pltpu.SemaphoreType.DMA((N_DEV - 1,)),
            pltpu.SemaphoreType.DMA((N_DEV - 1,)),
        ],
        compiler_params=pltpu.CompilerParams(collective_id=0),
    )(x, k, Wp)


# baseline (device time: 104817 ns/iter reference)
import jax
import jax.numpy as jnp
from jax import lax
from jax.experimental import pallas as pl
from jax.experimental.pallas import tpu as pltpu

N_DEV = 8


def kernel(x, k, Wp):
    B, S, C = x.shape
    T = k.shape[0]
    P = Wp.shape[1]

    def body(x_ref, k_ref, w_ref, out_ref, acc_ref, comm_ref, send_sems, recv_sems):
        my = lax.axis_index("i")
        right = lax.rem(my + 1, N_DEV)

        xv = x_ref[:, :, :]
        kv = k_ref[:, :]
        conv = xv * kv[T - 1][None, None, :]
        for t in range(T - 1):
            d = T - 1 - t
            shifted = jnp.concatenate(
                [jnp.zeros((B, d, C), dtype=xv.dtype), xv[:, : S - d, :]],
                axis=1,
            )
            conv = conv + shifted * kv[t][None, None, :]
        a = conv * jax.nn.sigmoid(conv)
        a2 = a.reshape(B * S, C).astype(jnp.bfloat16)
        w = w_ref[:, :].astype(jnp.bfloat16)
        partial = jnp.dot(a2, w, preferred_element_type=jnp.float32)

        acc_ref[:, :] = partial
        comm_ref[0, :, :] = partial.astype(jnp.bfloat16)

        for h in range(N_DEV - 1):
            rdma = pltpu.make_async_remote_copy(
                src_ref=comm_ref.at[h],
                dst_ref=comm_ref.at[h + 1],
                send_sem=send_sems.at[h],
                recv_sem=recv_sems.at[h],
                device_id=(right,),
                device_id_type=pl.DeviceIdType.MESH,
            )
            rdma.start()
            rdma.wait()
            acc_ref[:, :] = acc_ref[:, :] + comm_ref[h + 1, :, :].astype(jnp.float32)

        out_ref[:, :, :] = acc_ref[:, :].reshape(B, S, P)

    return pl.pallas_call(
        body,
        out_shape=jax.ShapeDtypeStruct((B, S, P), jnp.float32),
        in_specs=[
            pl.BlockSpec(memory_space=pltpu.VMEM),
            pl.BlockSpec(memory_space=pltpu.VMEM),
            pl.BlockSpec(memory_space=pltpu.VMEM),
        ],
        out_specs=pl.BlockSpec(memory_space=pltpu.VMEM),
        scratch_shapes=[
            pltpu.VMEM((B * S, P), jnp.float32),
            pltpu.VMEM((N_DEV, B * S, P), jnp.bfloat16),
            pltpu.SemaphoreType.DMA((N_DEV - 1,)),
            pltpu.SemaphoreType.DMA((N_DEV - 1,)),
        ],
    )(x, k, Wp)


# device time: 29978 ns/iter; 3.4965x vs baseline; 3.4965x over previous
import jax
import jax.numpy as jnp
from jax import lax
from jax.experimental import pallas as pl
from jax.experimental.pallas import tpu as pltpu

N_DEV = 8


def kernel(x, k, Wp):
    B, S, C = x.shape
    T = k.shape[0]
    P = Wp.shape[1]
    R = (B * S) // N_DEV

    CH_PER_B = N_DEV // B

    def body(x_ref, k_ref, w_ref, out_ref, part_ref, red_ref, p2src_ref,
             p1_buf, p2_buf, p1_send, p1_recv, p2_send, p2_recv):
        my = lax.axis_index("i")

        xv = x_ref[:, :, :]
        kv = k_ref[:, :]
        conv = xv * kv[T - 1][None, None, :]
        for t in range(T - 1):
            d = T - 1 - t
            shifted = jnp.concatenate(
                [jnp.zeros((B, d, C), dtype=xv.dtype), xv[:, : S - d, :]],
                axis=1,
            )
            conv = conv + shifted * kv[t][None, None, :]
        a = conv * jax.nn.sigmoid(conv)
        a2 = a.reshape(B * S, C).astype(jnp.bfloat16)
        w = w_ref[:, :].astype(jnp.bfloat16)
        partial = jnp.dot(a2, w, preferred_element_type=jnp.float32)
        part_ref[:, :, :] = partial.astype(jnp.bfloat16).reshape(N_DEV, R, P)

        for j in range(N_DEV):
            @pl.when(j != my)
            def _(j=j):
                rdma = pltpu.make_async_remote_copy(
                    src_ref=part_ref.at[j],
                    dst_ref=p1_buf.at[my],
                    send_sem=p1_send.at[j],
                    recv_sem=p1_recv.at[my],
                    device_id=(j,),
                    device_id_type=pl.DeviceIdType.MESH,
                )
                rdma.start()

            @pl.when(j == my)
            def _(j=j):
                p1_buf[j, :, :] = part_ref[j, :, :]

        for j in range(N_DEV):
            @pl.when(j != my)
            def _(j=j):
                rdma = pltpu.make_async_remote_copy(
                    src_ref=part_ref.at[j],
                    dst_ref=p1_buf.at[j],
                    send_sem=p1_send.at[j],
                    recv_sem=p1_recv.at[j],
                    device_id=(j,),
                    device_id_type=pl.DeviceIdType.MESH,
                )
                rdma.wait_recv()

        red_ref[:, :] = jnp.sum(
            p1_buf[:, :, :].astype(jnp.float32), axis=0
        )
        p2src_ref[:, :] = red_ref[:, :].astype(jnp.bfloat16)

        for j in range(N_DEV):
            @pl.when(j != my)
            def _(j=j):
                rdma = pltpu.make_async_remote_copy(
                    src_ref=p2src_ref,
                    dst_ref=p2_buf.at[my],
                    send_sem=p2_send.at[j],
                    recv_sem=p2_recv.at[my],
                    device_id=(j,),
                    device_id_type=pl.DeviceIdType.MESH,
                )
                rdma.start()

        for j in range(N_DEV):
            b = j // CH_PER_B
            s0 = (j % CH_PER_B) * R

            @pl.when(j != my)
            def _(j=j, b=b, s0=s0):
                rdma = pltpu.make_async_remote_copy(
                    src_ref=p2src_ref,
                    dst_ref=p2_buf.at[j],
                    send_sem=p2_send.at[j],
                    recv_sem=p2_recv.at[j],
                    device_id=(j,),
                    device_id_type=pl.DeviceIdType.MESH,
                )
                rdma.wait_recv()
                out_ref[b, pl.ds(s0, R), :] = p2_buf[j, :, :].astype(
                    jnp.float32
                )

            @pl.when(j == my)
            def _(j=j, b=b, s0=s0):
                out_ref[b, pl.ds(s0, R), :] = red_ref[:, :]

        for j in range(N_DEV):
            @pl.when(j != my)
            def _(j=j):
                s1 = pltpu.make_async_remote_copy(
                    src_ref=part_ref.at[j],
                    dst_ref=p1_buf.at[j],
                    send_sem=p1_send.at[j],
                    recv_sem=p1_recv.at[j],
                    device_id=(j,),
                    device_id_type=pl.DeviceIdType.MESH,
                )
                s1.wait_send()
                s2 = pltpu.make_async_remote_copy(
                    src_ref=p2src_ref,
                    dst_ref=p2_buf.at[j],
                    send_sem=p2_send.at[j],
                    recv_sem=p2_recv.at[j],
                    device_id=(j,),
                    device_id_type=pl.DeviceIdType.MESH,
                )
                s2.wait_send()

    return pl.pallas_call(
        body,
        out_shape=jax.ShapeDtypeStruct((B, S, P), jnp.float32),
        in_specs=[
            pl.BlockSpec(memory_space=pltpu.VMEM),
            pl.BlockSpec(memory_space=pltpu.VMEM),
            pl.BlockSpec(memory_space=pltpu.VMEM),
        ],
        out_specs=pl.BlockSpec(memory_space=pltpu.VMEM),
        scratch_shapes=[
            pltpu.VMEM((N_DEV, R, P), jnp.bfloat16),
            pltpu.VMEM((R, P), jnp.float32),
            pltpu.VMEM((R, P), jnp.bfloat16),
            pltpu.VMEM((N_DEV, R, P), jnp.bfloat16),
            pltpu.VMEM((N_DEV, R, P), jnp.bfloat16),
            pltpu.SemaphoreType.DMA((N_DEV,)),
            pltpu.SemaphoreType.DMA((N_DEV,)),
            pltpu.SemaphoreType.DMA((N_DEV,)),
            pltpu.SemaphoreType.DMA((N_DEV,)),
        ],
    )(x, k, Wp)


# device time: 29850 ns/iter; 3.5115x vs baseline; 1.0043x over previous
import jax
import jax.numpy as jnp
from jax import lax
from jax.experimental import pallas as pl
from jax.experimental.pallas import tpu as pltpu

N_DEV = 8


def kernel(x, k, Wp):
    B, S, C = x.shape
    T = k.shape[0]
    P = Wp.shape[1]
    R = (B * S) // N_DEV

    CH_PER_B = N_DEV // B

    def body(x_ref, k_ref, w_ref, out_ref, part_ref, red_ref, p2src_ref,
             p1_buf, p2_buf, p1_send, p1_recv, p2_send, p2_recv):
        my = lax.axis_index("i")

        xv = x_ref[:, :, :].astype(jnp.bfloat16)
        kv = k_ref[:, :].astype(jnp.bfloat16)
        conv = xv * kv[T - 1][None, None, :]
        for t in range(T - 1):
            d = T - 1 - t
            shifted = jnp.concatenate(
                [jnp.zeros((B, d, C), dtype=xv.dtype), xv[:, : S - d, :]],
                axis=1,
            )
            conv = conv + shifted * kv[t][None, None, :]
        a2 = (conv * jax.nn.sigmoid(conv)).reshape(B * S, C)
        w = w_ref[:, :].astype(jnp.bfloat16)

        for j in range(N_DEV):
            pj = jnp.dot(
                a2[j * R : (j + 1) * R, :], w,
                preferred_element_type=jnp.float32,
            )
            part_ref[j, :, :] = pj.astype(jnp.bfloat16)

            @pl.when(j != my)
            def _(j=j):
                rdma = pltpu.make_async_remote_copy(
                    src_ref=part_ref.at[j],
                    dst_ref=p1_buf.at[my],
                    send_sem=p1_send.at[j],
                    recv_sem=p1_recv.at[my],
                    device_id=(j,),
                    device_id_type=pl.DeviceIdType.MESH,
                )
                rdma.start()

            @pl.when(j == my)
            def _(j=j):
                p1_buf[j, :, :] = part_ref[j, :, :]

        for j in range(N_DEV):
            @pl.when(j != my)
            def _(j=j):
                rdma = pltpu.make_async_remote_copy(
                    src_ref=part_ref.at[j],
                    dst_ref=p1_buf.at[j],
                    send_sem=p1_send.at[j],
                    recv_sem=p1_recv.at[j],
                    device_id=(j,),
                    device_id_type=pl.DeviceIdType.MESH,
                )
                rdma.wait_recv()

        red_ref[:, :] = jnp.sum(
            p1_buf[:, :, :].astype(jnp.float32), axis=0
        )
        p2src_ref[:, :] = red_ref[:, :].astype(jnp.bfloat16)

        for j in range(N_DEV):
            @pl.when(j != my)
            def _(j=j):
                rdma = pltpu.make_async_remote_copy(
                    src_ref=p2src_ref,
                    dst_ref=p2_buf.at[my],
                    send_sem=p2_send.at[j],
                    recv_sem=p2_recv.at[my],
                    device_id=(j,),
                    device_id_type=pl.DeviceIdType.MESH,
                )
                rdma.start()

        for j in range(N_DEV):
            b = j // CH_PER_B
            s0 = (j % CH_PER_B) * R

            @pl.when(j != my)
            def _(j=j, b=b, s0=s0):
                rdma = pltpu.make_async_remote_copy(
                    src_ref=p2src_ref,
                    dst_ref=p2_buf.at[j],
                    send_sem=p2_send.at[j],
                    recv_sem=p2_recv.at[j],
                    device_id=(j,),
                    device_id_type=pl.DeviceIdType.MESH,
                )
                rdma.wait_recv()
                out_ref[b, pl.ds(s0, R), :] = p2_buf[j, :, :].astype(
                    jnp.float32
                )

            @pl.when(j == my)
            def _(j=j, b=b, s0=s0):
                out_ref[b, pl.ds(s0, R), :] = red_ref[:, :]

        for j in range(N_DEV):
            @pl.when(j != my)
            def _(j=j):
                s1 = pltpu.make_async_remote_copy(
                    src_ref=part_ref.at[j],
                    dst_ref=p1_buf.at[j],
                    send_sem=p1_send.at[j],
                    recv_sem=p1_recv.at[j],
                    device_id=(j,),
                    device_id_type=pl.DeviceIdType.MESH,
                )
                s1.wait_send()
                s2 = pltpu.make_async_remote_copy(
                    src_ref=p2src_ref,
                    dst_ref=p2_buf.at[j],
                    send_sem=p2_send.at[j],
                    recv_sem=p2_recv.at[j],
                    device_id=(j,),
                    device_id_type=pl.DeviceIdType.MESH,
                )
                s2.wait_send()

    return pl.pallas_call(
        body,
        out_shape=jax.ShapeDtypeStruct((B, S, P), jnp.float32),
        in_specs=[
            pl.BlockSpec(memory_space=pltpu.VMEM),
            pl.BlockSpec(memory_space=pltpu.VMEM),
            pl.BlockSpec(memory_space=pltpu.VMEM),
        ],
        out_specs=pl.BlockSpec(memory_space=pltpu.VMEM),
        scratch_shapes=[
            pltpu.VMEM((N_DEV, R, P), jnp.bfloat16),
            pltpu.VMEM((R, P), jnp.float32),
            pltpu.VMEM((R, P), jnp.bfloat16),
            pltpu.VMEM((N_DEV, R, P), jnp.bfloat16),
            pltpu.VMEM((N_DEV, R, P), jnp.bfloat16),
            pltpu.SemaphoreType.DMA((N_DEV,)),
            pltpu.SemaphoreType.DMA((N_DEV,)),
            pltpu.SemaphoreType.DMA((N_DEV,)),
            pltpu.SemaphoreType.DMA((N_DEV,)),
        ],
    )(x, k, Wp)
